# baseline (speedup 1.0000x reference)
import functools

import jax
import jax.numpy as jnp
from jax import lax
from jax.experimental import pallas as pl
from jax.experimental.pallas import tpu as pltpu
from jax.experimental.pallas import tpu_sc as plsc

DIM = 64
LANES = 16
NC = 2
NS = 16
NW = NC * NS
CHUNK = 128


def _two_tower_call(u3, i3, user_table, item_table, *, b_per_w, n_chunks):
    mesh = plsc.VectorSubcoreMesh(core_axis_name="c", subcore_axis_name="s")
    batch = b_per_w * NW

    @functools.partial(
        pl.kernel,
        mesh=mesh,
        compiler_params=pltpu.CompilerParams(
            needs_layout_passes=False, use_tc_tiling_on_sc=False),
        out_type=jax.ShapeDtypeStruct((batch,), jnp.float32),
        scratch_types=[
            pltpu.VMEM((n_chunks, CHUNK), jnp.int32),
            pltpu.VMEM((n_chunks, CHUNK), jnp.int32),
            pltpu.VMEM((b_per_w, DIM), jnp.float32),
            pltpu.VMEM((b_per_w, DIM), jnp.float32),
            pltpu.VMEM((b_per_w,), jnp.float32),
            pltpu.VMEM((LANES * LANES,), jnp.float32),
            pltpu.SemaphoreType.DMA,
        ],
    )
    def two_tower(u_hbm, i_hbm, ut_hbm, it_hbm, out_hbm,
                  u_idx, i_idx, u_rows, i_rows, out_v, t_scratch, sem):
        wid = lax.axis_index("s") * NC + lax.axis_index("c")
        pltpu.sync_copy(u_hbm.at[wid], u_idx)
        pltpu.sync_copy(i_hbm.at[wid], i_idx)
        copies = []
        for c in range(n_chunks):
            copies.append(pltpu.async_copy(
                ut_hbm.at[u_idx.at[c]],
                u_rows.at[pl.ds(c * CHUNK, CHUNK)], sem))
            copies.append(pltpu.async_copy(
                it_hbm.at[i_idx.at[c]],
                i_rows.at[pl.ds(c * CHUNK, CHUNK)], sem))
        for cp in copies:
            cp.wait()

        lanes = lax.broadcasted_iota(jnp.int32, (LANES,), 0)

        def body(g, carry):
            base = g * LANES
            out16 = jnp.zeros((LANES,), jnp.float32)
            for r in range(LANES):
                row = base + r
                acc = jnp.zeros((LANES,), jnp.float32)
                for k in range(DIM // LANES):
                    uu = u_rows[row, pl.ds(k * LANES, LANES)]
                    vv = i_rows[row, pl.ds(k * LANES, LANES)]
                    acc = acc + uu * vv
                total = jnp.sum(acc)
                out16 = jnp.where(lanes == r, total, out16)
            out_v[pl.ds(base, LANES)] = out16
            return carry

        lax.fori_loop(0, b_per_w // LANES, body, 0)
        pltpu.sync_copy(out_v, out_hbm.at[pl.ds(wid * b_per_w, b_per_w)])

    return two_tower(u3, i3, user_table, item_table)


def kernel(u, i, user_table, item_table):
    batch = u.shape[0]
    b_per_w = batch // NW
    n_chunks = b_per_w // CHUNK
    u3 = u.astype(jnp.int32).reshape(NW, n_chunks, CHUNK)
    i3 = i.astype(jnp.int32).reshape(NW, n_chunks, CHUNK)
    return _two_tower_call(u3, i3, user_table, item_table,
                           b_per_w=b_per_w, n_chunks=n_chunks)


# speedup vs baseline: 3.6610x; 3.6610x over previous
import functools

import jax
import jax.numpy as jnp
from jax import lax
from jax.experimental import pallas as pl
from jax.experimental.pallas import tpu as pltpu
from jax.experimental.pallas import tpu_sc as plsc

DIM = 64
LANES = 16
NC = 2
NS = 16
NW = NC * NS
NBUF = 4
NROW = 1000000
TAILBASE = (NROW // 128) * 128
LASTJ = NROW // 128 - 1

_CPARAMS = pltpu.CompilerParams(needs_layout_passes=False)


def _splat(x):
    return jnp.full((LANES,), x, jnp.int32)


def _gather_rows_call(r_sorted, perm3, t_t, *, b_per_w):
    mesh = plsc.VectorSubcoreMesh(core_axis_name="c", subcore_axis_name="s")
    batch = b_per_w * NW
    n_grp = b_per_w // LANES

    @functools.partial(
        pl.kernel,
        mesh=mesh,
        compiler_params=_CPARAMS,
        out_type=jax.ShapeDtypeStruct((batch, 128), jnp.float32),
        scratch_types=[
            pltpu.VMEM((b_per_w + LANES,), jnp.int32),
            pltpu.VMEM((b_per_w,), jnp.int32),
            pltpu.VMEM((b_per_w,), jnp.int32),
            pltpu.VMEM((8, 128), jnp.int32),
            pltpu.VMEM((NBUF, DIM, 128), jnp.float32),
            pltpu.VMEM((b_per_w, 128), jnp.float32),
            pltpu.SemaphoreType.DMA((NBUF,)),
            pltpu.SemaphoreType.DMA,
        ],
    )
    def gather_rows(r_hbm, p_hbm, t_hbm, rows_hbm,
                    r_loc, ord_loc, jlist, perm_loc, ring, rows_loc,
                    sem, ssem):
        wid = lax.axis_index("s") * NC + lax.axis_index("c")
        base = wid * b_per_w
        lanes = lax.broadcasted_iota(jnp.int32, (LANES,), 0)
        r_loc[pl.ds(0, LANES)] = _splat(-(1 << 30))
        pltpu.sync_copy(r_hbm.at[pl.ds(base, b_per_w)],
                        r_loc.at[pl.ds(LANES, b_per_w)])
        pltpu.sync_copy(p_hbm.at[wid], perm_loc)

        def sread(ref, p):
            return plsc.load_gather(ref, [_splat(0) + p])[0]

        def pass_a(g, off):
            t0 = g * LANES
            jv = r_loc[pl.ds(t0 + LANES, LANES)] >> 7
            pv = plsc.load_gather(r_loc, [_splat(t0 + 15) + lanes]) >> 7
            ch = jv != pv
            cs = plsc.cumsum(ch.astype(jnp.int32))
            ordv = off + cs - 1
            ord_loc[pl.ds(t0, LANES)] = ordv
            plsc.store_scatter(jlist, [ordv], jv, mask=ch)
            return off + cs[15]

        n_j = lax.fori_loop(0, n_grp, pass_a, jnp.int32(0))

        def fetch(ordinal, slot):
            jp = jnp.minimum(sread(jlist, jnp.minimum(ordinal, n_j - 1)),
                             LASTJ)
            off128 = pl.multiple_of(jp << 7, 128)
            pltpu.async_copy(t_hbm.at[:, pl.ds(off128, 128)],
                             ring.at[slot], sem.at[slot])

        for s in range(NBUF):
            fetch(jnp.int32(s), s)
        pltpu.make_async_copy(t_hbm.at[:, pl.ds(0, 128)], ring.at[0],
                              sem.at[0]).wait()

        def pass_b(g, cur):
            t0 = g * LANES
            rv = r_loc[pl.ds(t0 + LANES, LANES)]
            ov = ord_loc[pl.ds(t0, LANES)]
            for k in range(LANES):
                o = ov[k]
                r = rv[k]

                @pl.when(o > cur)
                def _():
                    pltpu.make_async_copy(
                        t_hbm.at[:, pl.ds(0, 128)],
                        ring.at[o & (NBUF - 1)],
                        sem.at[o & (NBUF - 1)]).wait()
                    fetch(o + NBUF - 1, (o + NBUF - 1) & (NBUF - 1))

                col = _splat(r & 127)
                slot = _splat(o & (NBUF - 1))
                for c in range(DIM // LANES):
                    vals = plsc.load_gather(
                        ring, [slot, c * LANES + lanes, col])
                    rows_loc[t0 + k, pl.ds(c * LANES, LANES)] = vals
                cur = jnp.maximum(cur, o)
            return cur

        lax.fori_loop(0, n_grp, pass_b, jnp.int32(0))

        last_slot = (n_j - 1) & (NBUF - 1)
        for s in range(NBUF):
            @pl.when(jnp.int32(s) != last_slot)
            def _():
                pltpu.make_async_copy(t_hbm.at[:, pl.ds(0, 128)],
                                      ring.at[s], sem.at[s]).wait()

        n_ch = b_per_w // 128
        for c in range(n_ch):
            pltpu.async_copy(rows_loc.at[pl.ds(c * 128, 128)],
                             rows_hbm.at[perm_loc.at[c]], ssem)
        for c in range(n_ch):
            pltpu.make_async_copy(rows_loc.at[pl.ds(0, 128)],
                                  rows_hbm.at[perm_loc.at[0]], ssem).wait()

    return gather_rows(r_sorted, perm3, t_t)


def _dot_call(u, i, ue_buf, ie_buf, tail_u, tail_i, *, b_per_w):
    mesh = plsc.VectorSubcoreMesh(core_axis_name="c", subcore_axis_name="s")
    batch = b_per_w * NW
    half = b_per_w // 2

    @functools.partial(
        pl.kernel,
        mesh=mesh,
        compiler_params=_CPARAMS,
        out_type=jax.ShapeDtypeStruct((batch,), jnp.float32),
        scratch_types=[
            pltpu.VMEM((b_per_w,), jnp.int32),
            pltpu.VMEM((b_per_w,), jnp.int32),
            pltpu.VMEM((half, 128), jnp.float32),
            pltpu.VMEM((half, 128), jnp.float32),
            pltpu.VMEM((DIM, 128), jnp.float32),
            pltpu.VMEM((DIM, 128), jnp.float32),
            pltpu.VMEM((b_per_w,), jnp.float32),
        ],
    )
    def dot_k(u_hbm, i_hbm, ue_hbm, ie_hbm, tu_hbm, ti_hbm, out_hbm,
              u_loc, i_loc, ue_loc, ie_loc, tu_loc, ti_loc, out_v):
        wid = lax.axis_index("s") * NC + lax.axis_index("c")
        base = wid * b_per_w
        lanes = lax.broadcasted_iota(jnp.int32, (LANES,), 0)
        pltpu.sync_copy(u_hbm.at[pl.ds(base, b_per_w)], u_loc)
        pltpu.sync_copy(i_hbm.at[pl.ds(base, b_per_w)], i_loc)
        pltpu.sync_copy(tu_hbm, tu_loc)
        pltpu.sync_copy(ti_hbm, ti_loc)

        for h in range(2):
            pltpu.sync_copy(ue_hbm.at[pl.ds(base + h * half, half)], ue_loc)
            pltpu.sync_copy(ie_hbm.at[pl.ds(base + h * half, half)], ie_loc)

            def grp(g, carry, h=h):
                t0 = h * half + g * LANES
                row0 = g * LANES
                uv = u_loc[pl.ds(t0, LANES)]
                iv = i_loc[pl.ds(t0, LANES)]
                out16 = jnp.zeros((LANES,), jnp.float32)
                for k in range(LANES):
                    ur = uv[k]
                    ir = iv[k]
                    cu = _splat(jnp.clip(ur - TAILBASE, 0, 63))
                    ci = _splat(jnp.clip(ir - TAILBASE, 0, 63))
                    acc = jnp.zeros((LANES,), jnp.float32)
                    for c in range(DIM // LANES):
                        cl = c * LANES + lanes
                        uu = ue_loc[row0 + k, pl.ds(c * LANES, LANES)]
                        ii = ie_loc[row0 + k, pl.ds(c * LANES, LANES)]
                        uu = jnp.where(ur >= TAILBASE,
                                       plsc.load_gather(tu_loc, [cl, cu]),
                                       uu)
                        ii = jnp.where(ir >= TAILBASE,
                                       plsc.load_gather(ti_loc, [cl, ci]),
                                       ii)
                        acc = acc + uu * ii
                    total = jnp.sum(acc)
                    out16 = jnp.where(lanes == k, total, out16)
                out_v[pl.ds(t0, LANES)] = out16
                return carry

            lax.fori_loop(0, half // LANES, grp, 0)

        pltpu.sync_copy(out_v, out_hbm.at[pl.ds(base, b_per_w)])

    return dot_k(u, i, ue_buf, ie_buf, tail_u, tail_i)


def kernel(u, i, user_table, item_table):
    batch = u.shape[0]
    b_per_w = batch // NW
    u32 = u.astype(jnp.int32)
    i32 = i.astype(jnp.int32)
    iota = jnp.arange(batch, dtype=jnp.int32)
    ru, pu = lax.sort_key_val(u32, iota)
    ri, pi = lax.sort_key_val(i32, iota)

    def perm3(p):
        p4 = p.reshape(NW, b_per_w // 128, 128)
        return jnp.pad(p4, ((0, 0), (0, 8 - b_per_w // 128), (0, 0)))

    ut_t = user_table.T
    it_t = item_table.T
    tail_u = jnp.pad(user_table[TAILBASE:].T, ((0, 0), (0, 128 - (NROW - TAILBASE))))
    tail_i = jnp.pad(item_table[TAILBASE:].T, ((0, 0), (0, 128 - (NROW - TAILBASE))))

    ue_buf = _gather_rows_call(ru, perm3(pu), ut_t, b_per_w=b_per_w)
    ie_buf = _gather_rows_call(ri, perm3(pi), it_t, b_per_w=b_per_w)
    return _dot_call(u32, i32, ue_buf, ie_buf, tail_u, tail_i,
                     b_per_w=b_per_w)


# speedup vs baseline: 4.1947x; 1.1458x over previous
import functools

import jax
import jax.numpy as jnp
from jax import lax
from jax.experimental import pallas as pl
from jax.experimental.pallas import tpu as pltpu
from jax.experimental.pallas import tpu_sc as plsc

DIM = 64
LANES = 16
NC = 2
NS = 16
NW = NC * NS
NBUF = 4
NROW = 1000000
TAILBASE = (NROW // 128) * 128
LASTJ = NROW // 128 - 1

_CPARAMS = pltpu.CompilerParams(needs_layout_passes=False)


def _splat(x):
    return jnp.full((LANES,), x, jnp.int32)


def _gather_rows_call(r_sorted, perm3, t_t, *, b_per_w):
    mesh = plsc.VectorSubcoreMesh(core_axis_name="c", subcore_axis_name="s")
    batch = b_per_w * NW
    n_grp = b_per_w // LANES

    @functools.partial(
        pl.kernel,
        mesh=mesh,
        compiler_params=_CPARAMS,
        out_type=jax.ShapeDtypeStruct((batch, 128), jnp.float32),
        scratch_types=[
            pltpu.VMEM((b_per_w + LANES,), jnp.int32),
            pltpu.VMEM((b_per_w,), jnp.int32),
            pltpu.VMEM((b_per_w,), jnp.int32),
            pltpu.VMEM((8, 128), jnp.int32),
            pltpu.VMEM((NBUF, DIM, 128), jnp.float32),
            pltpu.VMEM((b_per_w, 128), jnp.float32),
            pltpu.SemaphoreType.DMA((NBUF,)),
            pltpu.SemaphoreType.DMA,
        ],
    )
    def gather_rows(r_hbm, p_hbm, t_hbm, rows_hbm,
                    r_loc, ord_loc, jlist, perm_loc, ring, rows_loc,
                    sem, ssem):
        wid = lax.axis_index("s") * NC + lax.axis_index("c")
        base = wid * b_per_w
        lanes = lax.broadcasted_iota(jnp.int32, (LANES,), 0)
        r_loc[pl.ds(0, LANES)] = _splat(-(1 << 30))
        pltpu.sync_copy(r_hbm.at[pl.ds(base, b_per_w)],
                        r_loc.at[pl.ds(LANES, b_per_w)])
        pltpu.sync_copy(p_hbm.at[wid], perm_loc)

        def sread(ref, p):
            return plsc.load_gather(ref, [_splat(0) + p])[0]

        def pass_a(g, off):
            t0 = g * LANES
            jv = r_loc[pl.ds(t0 + LANES, LANES)] >> 7
            pv = plsc.load_gather(r_loc, [_splat(t0 + 15) + lanes]) >> 7
            ch = jv != pv
            cs = plsc.cumsum(ch.astype(jnp.int32))
            ordv = off + cs - 1
            ord_loc[pl.ds(t0, LANES)] = ordv
            plsc.store_scatter(jlist, [ordv], jv, mask=ch)
            return off + cs[15]

        n_j = lax.fori_loop(0, n_grp, pass_a, jnp.int32(0))

        def fetch(ordinal, slot):
            jp = jnp.minimum(sread(jlist, jnp.minimum(ordinal, n_j - 1)),
                             LASTJ)
            off128 = pl.multiple_of(jp << 7, 128)
            pltpu.async_copy(t_hbm.at[:, pl.ds(off128, 128)],
                             ring.at[slot], sem.at[slot])

        for s in range(NBUF):
            fetch(jnp.int32(s), s)
        pltpu.make_async_copy(t_hbm.at[:, pl.ds(0, 128)], ring.at[0],
                              sem.at[0]).wait()

        def pass_b(g, cur):
            t0 = g * LANES
            rv = r_loc[pl.ds(t0 + LANES, LANES)]
            ov = ord_loc[pl.ds(t0, LANES)]
            for k in range(LANES):
                o = ov[k]
                r = rv[k]

                @pl.when(o > cur)
                def _():
                    pltpu.make_async_copy(
                        t_hbm.at[:, pl.ds(0, 128)],
                        ring.at[o & (NBUF - 1)],
                        sem.at[o & (NBUF - 1)]).wait()
                    fetch(o + NBUF - 1, (o + NBUF - 1) & (NBUF - 1))

                col = _splat(r & 127)
                slot = _splat(o & (NBUF - 1))
                for c in range(DIM // LANES):
                    vals = plsc.load_gather(
                        ring, [slot, c * LANES + lanes, col])
                    rows_loc[t0 + k, pl.ds(c * LANES, LANES)] = vals
                cur = jnp.maximum(cur, o)
            return cur

        lax.fori_loop(0, n_grp, pass_b, jnp.int32(0))

        last_slot = (n_j - 1) & (NBUF - 1)
        for s in range(NBUF):
            @pl.when(jnp.int32(s) != last_slot)
            def _():
                pltpu.make_async_copy(t_hbm.at[:, pl.ds(0, 128)],
                                      ring.at[s], sem.at[s]).wait()

        n_ch = b_per_w // 128
        for c in range(n_ch):
            pltpu.async_copy(rows_loc.at[pl.ds(c * 128, 128)],
                             rows_hbm.at[perm_loc.at[c]], ssem)
        for c in range(n_ch):
            pltpu.make_async_copy(rows_loc.at[pl.ds(0, 128)],
                                  rows_hbm.at[perm_loc.at[0]], ssem).wait()

    return gather_rows(r_sorted, perm3, t_t)


def _dot_call(u, i, ue_buf, ie_buf, tail_u, tail_i, *, b_per_w):
    mesh = plsc.VectorSubcoreMesh(core_axis_name="c", subcore_axis_name="s")
    batch = b_per_w * NW
    half = b_per_w // 2

    @functools.partial(
        pl.kernel,
        mesh=mesh,
        compiler_params=_CPARAMS,
        out_type=jax.ShapeDtypeStruct((batch,), jnp.float32),
        scratch_types=[
            pltpu.VMEM((b_per_w,), jnp.int32),
            pltpu.VMEM((b_per_w,), jnp.int32),
            pltpu.VMEM((half, 128), jnp.float32),
            pltpu.VMEM((half, 128), jnp.float32),
            pltpu.VMEM((DIM, 128), jnp.float32),
            pltpu.VMEM((DIM, 128), jnp.float32),
            pltpu.VMEM((b_per_w,), jnp.float32),
            pltpu.SemaphoreType.DMA,
        ],
    )
    def dot_k(u_hbm, i_hbm, ue_hbm, ie_hbm, tu_hbm, ti_hbm, out_hbm,
              u_loc, i_loc, ue_loc, ie_loc, tu_loc, ti_loc, out_v, sem):
        wid = lax.axis_index("s") * NC + lax.axis_index("c")
        base = wid * b_per_w
        lanes = lax.broadcasted_iota(jnp.int32, (LANES,), 0)
        pltpu.sync_copy(u_hbm.at[pl.ds(base, b_per_w)], u_loc)
        pltpu.sync_copy(i_hbm.at[pl.ds(base, b_per_w)], i_loc)
        pltpu.sync_copy(tu_hbm, tu_loc)
        pltpu.sync_copy(ti_hbm, ti_loc)

        def stage(h):
            pltpu.async_copy(ue_hbm.at[pl.ds(base + h * half, half)],
                             ue_loc, sem)
            pltpu.async_copy(ie_hbm.at[pl.ds(base + h * half, half)],
                             ie_loc, sem)
            pltpu.make_async_copy(ue_hbm.at[pl.ds(0, half)], ue_loc,
                                  sem).wait()
            pltpu.make_async_copy(ie_hbm.at[pl.ds(0, half)], ie_loc,
                                  sem).wait()

        for h in range(2):
            stage(h)

            def grp(g, carry, h=h):
                t0 = h * half + g * LANES
                row0 = g * LANES
                uv = u_loc[pl.ds(t0, LANES)]
                iv = i_loc[pl.ds(t0, LANES)]
                out16 = jnp.zeros((LANES,), jnp.float32)
                for k in range(LANES):
                    acc = jnp.zeros((LANES,), jnp.float32)
                    for c in range(DIM // LANES):
                        uu = ue_loc[row0 + k, pl.ds(c * LANES, LANES)]
                        ii = ie_loc[row0 + k, pl.ds(c * LANES, LANES)]
                        acc = acc + uu * ii
                    total = jnp.sum(acc)
                    out16 = jnp.where(lanes == k, total, out16)
                out_v[pl.ds(t0, LANES)] = out16

                tail = (uv >= TAILBASE) | (iv >= TAILBASE)

                @pl.when(jnp.any(tail))
                def _():
                    out16s = out_v[pl.ds(t0, LANES)]
                    for k in range(LANES):
                        ur = uv[k]
                        ir = iv[k]
                        cu = _splat(jnp.clip(ur - TAILBASE, 0, 63))
                        ci = _splat(jnp.clip(ir - TAILBASE, 0, 63))
                        acc = jnp.zeros((LANES,), jnp.float32)
                        for c in range(DIM // LANES):
                            cl = c * LANES + lanes
                            uu = ue_loc[row0 + k, pl.ds(c * LANES, LANES)]
                            ii = ie_loc[row0 + k, pl.ds(c * LANES, LANES)]
                            uu = jnp.where(
                                ur >= TAILBASE,
                                plsc.load_gather(tu_loc, [cl, cu]), uu)
                            ii = jnp.where(
                                ir >= TAILBASE,
                                plsc.load_gather(ti_loc, [cl, ci]), ii)
                            acc = acc + uu * ii
                        total = jnp.sum(acc)
                        out16s = jnp.where(lanes == k, total, out16s)
                    out_v[pl.ds(t0, LANES)] = out16s

                return carry

            lax.fori_loop(0, half // LANES, grp, 0)

        pltpu.sync_copy(out_v, out_hbm.at[pl.ds(base, b_per_w)])

    return dot_k(u, i, ue_buf, ie_buf, tail_u, tail_i)


def kernel(u, i, user_table, item_table):
    batch = u.shape[0]
    b_per_w = batch // NW
    u32 = u.astype(jnp.int32)
    i32 = i.astype(jnp.int32)
    iota = jnp.arange(batch, dtype=jnp.int32)
    ru, pu = lax.sort_key_val(u32, iota)
    ri, pi = lax.sort_key_val(i32, iota)

    def perm3(p):
        p4 = p.reshape(NW, b_per_w // 128, 128)
        return jnp.pad(p4, ((0, 0), (0, 8 - b_per_w // 128), (0, 0)))

    ut_t = user_table.T
    it_t = item_table.T
    tail_u = jnp.pad(user_table[TAILBASE:].T, ((0, 0), (0, 128 - (NROW - TAILBASE))))
    tail_i = jnp.pad(item_table[TAILBASE:].T, ((0, 0), (0, 128 - (NROW - TAILBASE))))

    ue_buf = _gather_rows_call(ru, perm3(pu), ut_t, b_per_w=b_per_w)
    ie_buf = _gather_rows_call(ri, perm3(pi), it_t, b_per_w=b_per_w)
    return _dot_call(u32, i32, ue_buf, ie_buf, tail_u, tail_i,
                     b_per_w=b_per_w)


# speedup vs baseline: 4.2157x; 1.0050x over previous
import functools

import jax
import jax.numpy as jnp
from jax import lax
from jax.experimental import pallas as pl
from jax.experimental.pallas import tpu as pltpu
from jax.experimental.pallas import tpu_sc as plsc

DIM = 64
LANES = 16
NC = 2
NS = 16
NW = NC * NS
NBUF = 4
NROW = 1000000
TAILBASE = (NROW // 128) * 128
LASTJ = NROW // 128 - 1

_CPARAMS = pltpu.CompilerParams(needs_layout_passes=False)


def _splat(x):
    return jnp.full((LANES,), x, jnp.int32)


def _gather_rows_call(ru, permu3, ut_t, ri, permi3, it_t, *, b_per_w):
    mesh = plsc.VectorSubcoreMesh(core_axis_name="c", subcore_axis_name="s")
    batch = b_per_w * NW
    n_grp = b_per_w // LANES

    @functools.partial(
        pl.kernel,
        mesh=mesh,
        compiler_params=_CPARAMS,
        out_type=(jax.ShapeDtypeStruct((batch, 128), jnp.float32),
                  jax.ShapeDtypeStruct((batch, 128), jnp.float32)),
        scratch_types=[
            pltpu.VMEM((b_per_w + LANES,), jnp.int32),
            pltpu.VMEM((b_per_w,), jnp.int32),
            pltpu.VMEM((b_per_w,), jnp.int32),
            pltpu.VMEM((8, 128), jnp.int32),
            pltpu.VMEM((NBUF, DIM, 128), jnp.float32),
            pltpu.VMEM((b_per_w, 128), jnp.float32),
            pltpu.SemaphoreType.DMA((NBUF,)),
            pltpu.SemaphoreType.DMA,
        ],
    )
    def gather_rows(ru_hbm, pu_hbm, ut_hbm, ri_hbm, pi_hbm, it_hbm,
                    ue_hbm, ie_hbm,
                    r_loc, ord_loc, jlist, perm_loc, ring, rows_loc,
                    sem, ssem):
        wid = lax.axis_index("s") * NC + lax.axis_index("c")
        base = wid * b_per_w
        lanes = lax.broadcasted_iota(jnp.int32, (LANES,), 0)

        def sread(ref, p):
            return plsc.load_gather(ref, [_splat(0) + p])[0]

        def one_table(r_hbm, p_hbm, t_hbm, rows_hbm):
            r_loc[pl.ds(0, LANES)] = _splat(-(1 << 30))
            pltpu.sync_copy(r_hbm.at[pl.ds(base, b_per_w)],
                            r_loc.at[pl.ds(LANES, b_per_w)])
            pltpu.sync_copy(p_hbm.at[wid], perm_loc)

            def pass_a(g, off):
                t0 = g * LANES
                jv = r_loc[pl.ds(t0 + LANES, LANES)] >> 7
                pv = plsc.load_gather(r_loc, [_splat(t0 + 15) + lanes]) >> 7
                ch = jv != pv
                cs = plsc.cumsum(ch.astype(jnp.int32))
                ordv = off + cs - 1
                ord_loc[pl.ds(t0, LANES)] = ordv
                plsc.store_scatter(jlist, [ordv], jv, mask=ch)
                return off + cs[15]

            n_j = lax.fori_loop(0, n_grp, pass_a, jnp.int32(0))

            def fetch(ordinal, slot):
                jp = jnp.minimum(sread(jlist, jnp.minimum(ordinal, n_j - 1)),
                                 LASTJ)
                off128 = pl.multiple_of(jp << 7, 128)
                pltpu.async_copy(t_hbm.at[:, pl.ds(off128, 128)],
                                 ring.at[slot], sem.at[slot])

            for s in range(NBUF):
                fetch(jnp.int32(s), s)
            pltpu.make_async_copy(t_hbm.at[:, pl.ds(0, 128)], ring.at[0],
                                  sem.at[0]).wait()

            def pass_b(g, cur):
                t0 = g * LANES
                rv = r_loc[pl.ds(t0 + LANES, LANES)]
                ov = ord_loc[pl.ds(t0, LANES)]
                for k in range(LANES):
                    o = ov[k]
                    r = rv[k]

                    @pl.when(o > cur)
                    def _():
                        pltpu.make_async_copy(
                            t_hbm.at[:, pl.ds(0, 128)],
                            ring.at[o & (NBUF - 1)],
                            sem.at[o & (NBUF - 1)]).wait()
                        fetch(o + NBUF - 1, (o + NBUF - 1) & (NBUF - 1))

                    col = _splat(r & 127)
                    slot = _splat(o & (NBUF - 1))
                    for c in range(DIM // LANES):
                        vals = plsc.load_gather(
                            ring, [slot, c * LANES + lanes, col])
                        rows_loc[t0 + k, pl.ds(c * LANES, LANES)] = vals
                    cur = jnp.maximum(cur, o)
                return cur

            lax.fori_loop(0, n_grp, pass_b, jnp.int32(0))

            last_slot = (n_j - 1) & (NBUF - 1)
            for s in range(NBUF):
                @pl.when(jnp.int32(s) != last_slot)
                def _():
                    pltpu.make_async_copy(t_hbm.at[:, pl.ds(0, 128)],
                                          ring.at[s], sem.at[s]).wait()

            n_ch = b_per_w // 128
            for c in range(n_ch):
                pltpu.async_copy(rows_loc.at[pl.ds(c * 128, 128)],
                                 rows_hbm.at[perm_loc.at[c]], ssem)
            for c in range(n_ch):
                pltpu.make_async_copy(rows_loc.at[pl.ds(0, 128)],
                                     rows_hbm.at[perm_loc.at[0]],
                                     ssem).wait()

        one_table(ru_hbm, pu_hbm, ut_hbm, ue_hbm)
        one_table(ri_hbm, pi_hbm, it_hbm, ie_hbm)

    return gather_rows(ru, permu3, ut_t, ri, permi3, it_t)


def _dot_call(u, i, ue_buf, ie_buf, tail_u, tail_i, *, b_per_w):
    mesh = plsc.VectorSubcoreMesh(core_axis_name="c", subcore_axis_name="s")
    batch = b_per_w * NW
    half = b_per_w // 2

    @functools.partial(
        pl.kernel,
        mesh=mesh,
        compiler_params=_CPARAMS,
        out_type=jax.ShapeDtypeStruct((batch,), jnp.float32),
        scratch_types=[
            pltpu.VMEM((b_per_w,), jnp.int32),
            pltpu.VMEM((b_per_w,), jnp.int32),
            pltpu.VMEM((half, 128), jnp.float32),
            pltpu.VMEM((half, 128), jnp.float32),
            pltpu.VMEM((DIM, 128), jnp.float32),
            pltpu.VMEM((DIM, 128), jnp.float32),
            pltpu.VMEM((b_per_w,), jnp.float32),
            pltpu.SemaphoreType.DMA,
        ],
    )
    def dot_k(u_hbm, i_hbm, ue_hbm, ie_hbm, tu_hbm, ti_hbm, out_hbm,
              u_loc, i_loc, ue_loc, ie_loc, tu_loc, ti_loc, out_v, sem):
        wid = lax.axis_index("s") * NC + lax.axis_index("c")
        base = wid * b_per_w
        lanes = lax.broadcasted_iota(jnp.int32, (LANES,), 0)
        pltpu.sync_copy(u_hbm.at[pl.ds(base, b_per_w)], u_loc)
        pltpu.sync_copy(i_hbm.at[pl.ds(base, b_per_w)], i_loc)
        pltpu.sync_copy(tu_hbm, tu_loc)
        pltpu.sync_copy(ti_hbm, ti_loc)

        def stage(h):
            pltpu.async_copy(ue_hbm.at[pl.ds(base + h * half, half)],
                             ue_loc, sem)
            pltpu.async_copy(ie_hbm.at[pl.ds(base + h * half, half)],
                             ie_loc, sem)
            pltpu.make_async_copy(ue_hbm.at[pl.ds(0, half)], ue_loc,
                                  sem).wait()
            pltpu.make_async_copy(ie_hbm.at[pl.ds(0, half)], ie_loc,
                                  sem).wait()

        for h in range(2):
            stage(h)

            def grp(g, carry, h=h):
                t0 = h * half + g * LANES
                row0 = g * LANES
                uv = u_loc[pl.ds(t0, LANES)]
                iv = i_loc[pl.ds(t0, LANES)]
                out16 = jnp.zeros((LANES,), jnp.float32)
                for k in range(LANES):
                    acc = jnp.zeros((LANES,), jnp.float32)
                    for c in range(DIM // LANES):
                        uu = ue_loc[row0 + k, pl.ds(c * LANES, LANES)]
                        ii = ie_loc[row0 + k, pl.ds(c * LANES, LANES)]
                        acc = acc + uu * ii
                    total = jnp.sum(acc)
                    out16 = jnp.where(lanes == k, total, out16)
                out_v[pl.ds(t0, LANES)] = out16

                tail = (uv >= TAILBASE) | (iv >= TAILBASE)

                @pl.when(jnp.any(tail))
                def _():
                    out16s = out_v[pl.ds(t0, LANES)]
                    for k in range(LANES):
                        ur = uv[k]
                        ir = iv[k]
                        cu = _splat(jnp.clip(ur - TAILBASE, 0, 63))
                        ci = _splat(jnp.clip(ir - TAILBASE, 0, 63))
                        acc = jnp.zeros((LANES,), jnp.float32)
                        for c in range(DIM // LANES):
                            cl = c * LANES + lanes
                            uu = ue_loc[row0 + k, pl.ds(c * LANES, LANES)]
                            ii = ie_loc[row0 + k, pl.ds(c * LANES, LANES)]
                            uu = jnp.where(
                                ur >= TAILBASE,
                                plsc.load_gather(tu_loc, [cl, cu]), uu)
                            ii = jnp.where(
                                ir >= TAILBASE,
                                plsc.load_gather(ti_loc, [cl, ci]), ii)
                            acc = acc + uu * ii
                        total = jnp.sum(acc)
                        out16s = jnp.where(lanes == k, total, out16s)
                    out_v[pl.ds(t0, LANES)] = out16s

                return carry

            lax.fori_loop(0, half // LANES, grp, 0)

        pltpu.sync_copy(out_v, out_hbm.at[pl.ds(base, b_per_w)])

    return dot_k(u, i, ue_buf, ie_buf, tail_u, tail_i)


def kernel(u, i, user_table, item_table):
    batch = u.shape[0]
    b_per_w = batch // NW
    u32 = u.astype(jnp.int32)
    i32 = i.astype(jnp.int32)
    iota = jnp.arange(batch, dtype=jnp.int32)
    ru, pu = lax.sort_key_val(u32, iota)
    ri, pi = lax.sort_key_val(i32, iota)

    def perm3(p):
        p4 = p.reshape(NW, b_per_w // 128, 128)
        return jnp.pad(p4, ((0, 0), (0, 8 - b_per_w // 128), (0, 0)))

    ut_t = user_table.T
    it_t = item_table.T
    tail_u = jnp.pad(user_table[TAILBASE:].T, ((0, 0), (0, 128 - (NROW - TAILBASE))))
    tail_i = jnp.pad(item_table[TAILBASE:].T, ((0, 0), (0, 128 - (NROW - TAILBASE))))

    ue_buf, ie_buf = _gather_rows_call(ru, perm3(pu), ut_t,
                                       ri, perm3(pi), it_t,
                                       b_per_w=b_per_w)
    return _dot_call(u32, i32, ue_buf, ie_buf, tail_u, tail_i,
                     b_per_w=b_per_w)


# speedup vs baseline: 4.8075x; 1.1404x over previous
import functools

import jax
import jax.numpy as jnp
from jax import lax
from jax.experimental import pallas as pl
from jax.experimental.pallas import tpu as pltpu
from jax.experimental.pallas import tpu_sc as plsc

DIM = 64
LANES = 16
NC = 2
NS = 16
NW = NC * NS
NBUF = 6
NROW = 1000000
TAILBASE = (NROW // 128) * 128
LASTJ = NROW // 128 - 1

_CPARAMS = pltpu.CompilerParams(needs_layout_passes=False)


def _splat(x):
    return jnp.full((LANES,), x, jnp.int32)


def _gather_rows_call(ru, permu3, ut_t, ri, permi3, it_t, *, b_per_w):
    mesh = plsc.VectorSubcoreMesh(core_axis_name="c", subcore_axis_name="s")
    batch = b_per_w * NW
    n_grp = b_per_w // LANES

    @functools.partial(
        pl.kernel,
        mesh=mesh,
        compiler_params=_CPARAMS,
        out_type=(jax.ShapeDtypeStruct((batch, 128), jnp.float32),
                  jax.ShapeDtypeStruct((batch, 128), jnp.float32)),
        scratch_types=[
            pltpu.VMEM((b_per_w + LANES,), jnp.int32),
            pltpu.VMEM((b_per_w,), jnp.int32),
            pltpu.VMEM((b_per_w,), jnp.int32),
            pltpu.VMEM((8, 128), jnp.int32),
            pltpu.VMEM((NBUF, DIM, 128), jnp.float32),
            pltpu.VMEM((b_per_w, 128), jnp.float32),
            pltpu.SemaphoreType.DMA((NBUF,)),
            pltpu.SemaphoreType.DMA,
        ],
    )
    def gather_rows(ru_hbm, pu_hbm, ut_hbm, ri_hbm, pi_hbm, it_hbm,
                    ue_hbm, ie_hbm,
                    r_loc, ord_loc, jlist, perm_loc, ring, rows_loc,
                    sem, ssem):
        wid = lax.axis_index("s") * NC + lax.axis_index("c")
        base = wid * b_per_w
        lanes = lax.broadcasted_iota(jnp.int32, (LANES,), 0)

        def sread(ref, p):
            return plsc.load_gather(ref, [_splat(0) + p])[0]

        def one_table(r_hbm, p_hbm, t_hbm, rows_hbm):
            r_loc[pl.ds(0, LANES)] = _splat(-(1 << 30))
            pltpu.sync_copy(r_hbm.at[pl.ds(base, b_per_w)],
                            r_loc.at[pl.ds(LANES, b_per_w)])
            pltpu.sync_copy(p_hbm.at[wid], perm_loc)

            def pass_a(g, off):
                t0 = g * LANES
                jv = r_loc[pl.ds(t0 + LANES, LANES)] >> 7
                pv = plsc.load_gather(r_loc, [_splat(t0 + 15) + lanes]) >> 7
                ch = jv != pv
                cs = plsc.cumsum(ch.astype(jnp.int32))
                ordv = off + cs - 1
                ord_loc[pl.ds(t0, LANES)] = ordv
                plsc.store_scatter(jlist, [ordv], jv, mask=ch)
                return off + cs[15]

            n_j = lax.fori_loop(0, n_grp, pass_a, jnp.int32(0))

            def fetch(ordinal, slot):
                jp = jnp.minimum(sread(jlist, jnp.minimum(ordinal, n_j - 1)),
                                 LASTJ)
                off128 = pl.multiple_of(jp << 7, 128)
                pltpu.async_copy(t_hbm.at[:, pl.ds(off128, 128)],
                                 ring.at[slot], sem.at[slot])

            for s in range(NBUF):
                fetch(jnp.int32(s), s)
            pltpu.make_async_copy(t_hbm.at[:, pl.ds(0, 128)], ring.at[0],
                                  sem.at[0]).wait()

            def pass_b(g, cur):
                t0 = g * LANES
                rv = r_loc[pl.ds(t0 + LANES, LANES)]
                ov = ord_loc[pl.ds(t0, LANES)]
                for k in range(LANES):
                    o = ov[k]
                    r = rv[k]

                    @pl.when(o > cur)
                    def _():
                        pltpu.make_async_copy(
                            t_hbm.at[:, pl.ds(0, 128)],
                            ring.at[o % NBUF],
                            sem.at[o % NBUF]).wait()
                        fetch(o + NBUF - 1, (o + NBUF - 1) % NBUF)

                    col = _splat(r & 127)
                    slot = _splat(o % NBUF)
                    for c in range(DIM // LANES):
                        vals = plsc.load_gather(
                            ring, [slot, c * LANES + lanes, col])
                        rows_loc[t0 + k, pl.ds(c * LANES, LANES)] = vals
                    cur = jnp.maximum(cur, o)
                return cur

            lax.fori_loop(0, n_grp, pass_b, jnp.int32(0))

            last_slot = (n_j - 1) % NBUF
            for s in range(NBUF):
                @pl.when(jnp.int32(s) != last_slot)
                def _():
                    pltpu.make_async_copy(t_hbm.at[:, pl.ds(0, 128)],
                                          ring.at[s], sem.at[s]).wait()

            n_ch = b_per_w // 128
            for c in range(n_ch):
                pltpu.async_copy(rows_loc.at[pl.ds(c * 128, 128)],
                                 rows_hbm.at[perm_loc.at[c]], ssem)
            for c in range(n_ch):
                pltpu.make_async_copy(rows_loc.at[pl.ds(0, 128)],
                                     rows_hbm.at[perm_loc.at[0]],
                                     ssem).wait()

        one_table(ru_hbm, pu_hbm, ut_hbm, ue_hbm)
        one_table(ri_hbm, pi_hbm, it_hbm, ie_hbm)

    return gather_rows(ru, permu3, ut_t, ri, permi3, it_t)


def _dot_call(u, i, ue_buf, ie_buf, tail_u, tail_i, *, b_per_w):
    mesh = plsc.VectorSubcoreMesh(core_axis_name="c", subcore_axis_name="s")
    batch = b_per_w * NW
    half = b_per_w // 2

    @functools.partial(
        pl.kernel,
        mesh=mesh,
        compiler_params=_CPARAMS,
        out_type=jax.ShapeDtypeStruct((batch,), jnp.float32),
        scratch_types=[
            pltpu.VMEM((b_per_w,), jnp.int32),
            pltpu.VMEM((b_per_w,), jnp.int32),
            pltpu.VMEM((half, 128), jnp.float32),
            pltpu.VMEM((half, 128), jnp.float32),
            pltpu.VMEM((DIM, 128), jnp.float32),
            pltpu.VMEM((DIM, 128), jnp.float32),
            pltpu.VMEM((b_per_w,), jnp.float32),
            pltpu.SemaphoreType.DMA,
        ],
    )
    def dot_k(u_hbm, i_hbm, ue_hbm, ie_hbm, tu_hbm, ti_hbm, out_hbm,
              u_loc, i_loc, ue_loc, ie_loc, tu_loc, ti_loc, out_v, sem):
        wid = lax.axis_index("s") * NC + lax.axis_index("c")
        base = wid * b_per_w
        lanes = lax.broadcasted_iota(jnp.int32, (LANES,), 0)
        pltpu.sync_copy(u_hbm.at[pl.ds(base, b_per_w)], u_loc)
        pltpu.sync_copy(i_hbm.at[pl.ds(base, b_per_w)], i_loc)
        pltpu.sync_copy(tu_hbm, tu_loc)
        pltpu.sync_copy(ti_hbm, ti_loc)

        def stage(h):
            pltpu.async_copy(ue_hbm.at[pl.ds(base + h * half, half)],
                             ue_loc, sem)
            pltpu.async_copy(ie_hbm.at[pl.ds(base + h * half, half)],
                             ie_loc, sem)
            pltpu.make_async_copy(ue_hbm.at[pl.ds(0, half)], ue_loc,
                                  sem).wait()
            pltpu.make_async_copy(ie_hbm.at[pl.ds(0, half)], ie_loc,
                                  sem).wait()

        for h in range(2):
            stage(h)

            def grp(g, carry, h=h):
                t0 = h * half + g * LANES
                row0 = g * LANES
                uv = u_loc[pl.ds(t0, LANES)]
                iv = i_loc[pl.ds(t0, LANES)]
                out16 = jnp.zeros((LANES,), jnp.float32)
                for k in range(LANES):
                    acc = jnp.zeros((LANES,), jnp.float32)
                    for c in range(DIM // LANES):
                        uu = ue_loc[row0 + k, pl.ds(c * LANES, LANES)]
                        ii = ie_loc[row0 + k, pl.ds(c * LANES, LANES)]
                        acc = acc + uu * ii
                    total = jnp.sum(acc)
                    out16 = jnp.where(lanes == k, total, out16)
                out_v[pl.ds(t0, LANES)] = out16

                tail = (uv >= TAILBASE) | (iv >= TAILBASE)

                @pl.when(jnp.any(tail))
                def _():
                    out16s = out_v[pl.ds(t0, LANES)]
                    for k in range(LANES):
                        ur = uv[k]
                        ir = iv[k]
                        cu = _splat(jnp.clip(ur - TAILBASE, 0, 63))
                        ci = _splat(jnp.clip(ir - TAILBASE, 0, 63))
                        acc = jnp.zeros((LANES,), jnp.float32)
                        for c in range(DIM // LANES):
                            cl = c * LANES + lanes
                            uu = ue_loc[row0 + k, pl.ds(c * LANES, LANES)]
                            ii = ie_loc[row0 + k, pl.ds(c * LANES, LANES)]
                            uu = jnp.where(
                                ur >= TAILBASE,
                                plsc.load_gather(tu_loc, [cl, cu]), uu)
                            ii = jnp.where(
                                ir >= TAILBASE,
                                plsc.load_gather(ti_loc, [cl, ci]), ii)
                            acc = acc + uu * ii
                        total = jnp.sum(acc)
                        out16s = jnp.where(lanes == k, total, out16s)
                    out_v[pl.ds(t0, LANES)] = out16s

                return carry

            lax.fori_loop(0, half // LANES, grp, 0)

        pltpu.sync_copy(out_v, out_hbm.at[pl.ds(base, b_per_w)])

    return dot_k(u, i, ue_buf, ie_buf, tail_u, tail_i)


def kernel(u, i, user_table, item_table):
    batch = u.shape[0]
    b_per_w = batch // NW
    u32 = u.astype(jnp.int32)
    i32 = i.astype(jnp.int32)
    iota = jnp.arange(batch, dtype=jnp.int32)
    ru, pu = lax.sort_key_val(u32, iota)
    ri, pi = lax.sort_key_val(i32, iota)

    def perm3(p):
        p4 = p.reshape(NW, b_per_w // 128, 128)
        return jnp.pad(p4, ((0, 0), (0, 8 - b_per_w // 128), (0, 0)))

    ut_t = user_table.T
    it_t = item_table.T
    tail_u = jnp.pad(user_table[TAILBASE:].T, ((0, 0), (0, 128 - (NROW - TAILBASE))))
    tail_i = jnp.pad(item_table[TAILBASE:].T, ((0, 0), (0, 128 - (NROW - TAILBASE))))

    ue_buf, ie_buf = _gather_rows_call(ru, perm3(pu), ut_t,
                                       ri, perm3(pi), it_t,
                                       b_per_w=b_per_w)
    return _dot_call(u32, i32, ue_buf, ie_buf, tail_u, tail_i,
                     b_per_w=b_per_w)


# speedup vs baseline: 4.8782x; 1.0147x over previous
import functools

import jax
import jax.numpy as jnp
from jax import lax
from jax.experimental import pallas as pl
from jax.experimental.pallas import tpu as pltpu
from jax.experimental.pallas import tpu_sc as plsc

DIM = 64
LANES = 16
NC = 2
NS = 16
NW = NC * NS
NBUF = 10
NROW = 1000000
TAILBASE = (NROW // 128) * 128
LASTJ = NROW // 128 - 1

_CPARAMS = pltpu.CompilerParams(needs_layout_passes=False)


def _splat(x):
    return jnp.full((LANES,), x, jnp.int32)


def _gather_rows_call(ru, permu3, ut_t, ri, permi3, it_t, *, b_per_w):
    mesh = plsc.VectorSubcoreMesh(core_axis_name="c", subcore_axis_name="s")
    batch = b_per_w * NW
    n_grp = b_per_w // LANES

    @functools.partial(
        pl.kernel,
        mesh=mesh,
        compiler_params=_CPARAMS,
        out_type=(jax.ShapeDtypeStruct((batch, 128), jnp.float32),
                  jax.ShapeDtypeStruct((batch, 128), jnp.float32)),
        scratch_types=[
            pltpu.VMEM((b_per_w + LANES,), jnp.int32),
            pltpu.VMEM((b_per_w,), jnp.int32),
            pltpu.VMEM((b_per_w,), jnp.int32),
            pltpu.VMEM((8, 128), jnp.int32),
            pltpu.VMEM((NBUF, DIM, 128), jnp.float32),
            pltpu.VMEM((b_per_w // 2, 128), jnp.float32),
            pltpu.SemaphoreType.DMA((NBUF,)),
            pltpu.SemaphoreType.DMA,
        ],
    )
    def gather_rows(ru_hbm, pu_hbm, ut_hbm, ri_hbm, pi_hbm, it_hbm,
                    ue_hbm, ie_hbm,
                    r_loc, ord_loc, jlist, perm_loc, ring, rows_loc,
                    sem, ssem):
        wid = lax.axis_index("s") * NC + lax.axis_index("c")
        base = wid * b_per_w
        lanes = lax.broadcasted_iota(jnp.int32, (LANES,), 0)

        def sread(ref, p):
            return plsc.load_gather(ref, [_splat(0) + p])[0]

        def one_table(r_hbm, p_hbm, t_hbm, rows_hbm):
            r_loc[pl.ds(0, LANES)] = _splat(-(1 << 30))
            pltpu.sync_copy(r_hbm.at[pl.ds(base, b_per_w)],
                            r_loc.at[pl.ds(LANES, b_per_w)])
            pltpu.sync_copy(p_hbm.at[wid], perm_loc)

            def pass_a(g, off):
                t0 = g * LANES
                jv = r_loc[pl.ds(t0 + LANES, LANES)] >> 7
                pv = plsc.load_gather(r_loc, [_splat(t0 + 15) + lanes]) >> 7
                ch = jv != pv
                cs = plsc.cumsum(ch.astype(jnp.int32))
                ordv = off + cs - 1
                ord_loc[pl.ds(t0, LANES)] = ordv
                plsc.store_scatter(jlist, [ordv], jv, mask=ch)
                return off + cs[15]

            n_j = lax.fori_loop(0, n_grp, pass_a, jnp.int32(0))

            def fetch(ordinal, slot):
                jp = jnp.minimum(sread(jlist, jnp.minimum(ordinal, n_j - 1)),
                                 LASTJ)
                off128 = pl.multiple_of(jp << 7, 128)
                pltpu.async_copy(t_hbm.at[:, pl.ds(off128, 128)],
                                 ring.at[slot], sem.at[slot])

            for s in range(NBUF):
                fetch(jnp.int32(s), s)
            pltpu.make_async_copy(t_hbm.at[:, pl.ds(0, 128)], ring.at[0],
                                  sem.at[0]).wait()

            hgrp = n_grp // 2
            cur = jnp.int32(0)
            for hh in range(2):
                if hh == 1:
                    for _ in range(2):
                        pltpu.make_async_copy(
                            rows_loc.at[pl.ds(0, 128)],
                            rows_hbm.at[perm_loc.at[0]], ssem).wait()

                def pass_b(g, cur, hh=hh):
                    t0 = g * LANES
                    rl0 = t0 - hh * (b_per_w // 2)
                    rv = r_loc[pl.ds(t0 + LANES, LANES)]
                    ov = ord_loc[pl.ds(t0, LANES)]
                    for k in range(LANES):
                        o = ov[k]
                        r = rv[k]

                        @pl.when(o > cur)
                        def _():
                            pltpu.make_async_copy(
                                t_hbm.at[:, pl.ds(0, 128)],
                                ring.at[o % NBUF],
                                sem.at[o % NBUF]).wait()
                            fetch(o + NBUF - 1, (o + NBUF - 1) % NBUF)

                        col = _splat(r & 127)
                        slot = _splat(o % NBUF)
                        for c in range(DIM // LANES):
                            vals = plsc.load_gather(
                                ring, [slot, c * LANES + lanes, col])
                            rows_loc[rl0 + k, pl.ds(c * LANES, LANES)] = vals
                        cur = jnp.maximum(cur, o)
                    return cur

                cur = lax.fori_loop(hh * hgrp, (hh + 1) * hgrp, pass_b, cur)
                for c in range(2):
                    pltpu.async_copy(rows_loc.at[pl.ds(c * 128, 128)],
                                     rows_hbm.at[perm_loc.at[hh * 2 + c]],
                                     ssem)

            last_slot = (n_j - 1) % NBUF
            for s in range(NBUF):
                @pl.when(jnp.int32(s) != last_slot)
                def _():
                    pltpu.make_async_copy(t_hbm.at[:, pl.ds(0, 128)],
                                          ring.at[s], sem.at[s]).wait()

            for _ in range(2):
                pltpu.make_async_copy(rows_loc.at[pl.ds(0, 128)],
                                      rows_hbm.at[perm_loc.at[0]],
                                      ssem).wait()

        one_table(ru_hbm, pu_hbm, ut_hbm, ue_hbm)
        one_table(ri_hbm, pi_hbm, it_hbm, ie_hbm)

    return gather_rows(ru, permu3, ut_t, ri, permi3, it_t)


def _dot_call(u, i, ue_buf, ie_buf, tail_u, tail_i, *, b_per_w):
    mesh = plsc.VectorSubcoreMesh(core_axis_name="c", subcore_axis_name="s")
    batch = b_per_w * NW
    half = b_per_w // 2

    @functools.partial(
        pl.kernel,
        mesh=mesh,
        compiler_params=_CPARAMS,
        out_type=jax.ShapeDtypeStruct((batch,), jnp.float32),
        scratch_types=[
            pltpu.VMEM((b_per_w,), jnp.int32),
            pltpu.VMEM((b_per_w,), jnp.int32),
            pltpu.VMEM((half, 128), jnp.float32),
            pltpu.VMEM((half, 128), jnp.float32),
            pltpu.VMEM((DIM, 128), jnp.float32),
            pltpu.VMEM((DIM, 128), jnp.float32),
            pltpu.VMEM((b_per_w,), jnp.float32),
            pltpu.SemaphoreType.DMA,
        ],
    )
    def dot_k(u_hbm, i_hbm, ue_hbm, ie_hbm, tu_hbm, ti_hbm, out_hbm,
              u_loc, i_loc, ue_loc, ie_loc, tu_loc, ti_loc, out_v, sem):
        wid = lax.axis_index("s") * NC + lax.axis_index("c")
        base = wid * b_per_w
        lanes = lax.broadcasted_iota(jnp.int32, (LANES,), 0)
        pltpu.sync_copy(u_hbm.at[pl.ds(base, b_per_w)], u_loc)
        pltpu.sync_copy(i_hbm.at[pl.ds(base, b_per_w)], i_loc)
        pltpu.sync_copy(tu_hbm, tu_loc)
        pltpu.sync_copy(ti_hbm, ti_loc)

        def stage(h):
            pltpu.async_copy(ue_hbm.at[pl.ds(base + h * half, half)],
                             ue_loc, sem)
            pltpu.async_copy(ie_hbm.at[pl.ds(base + h * half, half)],
                             ie_loc, sem)
            pltpu.make_async_copy(ue_hbm.at[pl.ds(0, half)], ue_loc,
                                  sem).wait()
            pltpu.make_async_copy(ie_hbm.at[pl.ds(0, half)], ie_loc,
                                  sem).wait()

        for h in range(2):
            stage(h)

            def grp(g, carry, h=h):
                t0 = h * half + g * LANES
                row0 = g * LANES
                uv = u_loc[pl.ds(t0, LANES)]
                iv = i_loc[pl.ds(t0, LANES)]
                out16 = jnp.zeros((LANES,), jnp.float32)
                for k in range(LANES):
                    acc = jnp.zeros((LANES,), jnp.float32)
                    for c in range(DIM // LANES):
                        uu = ue_loc[row0 + k, pl.ds(c * LANES, LANES)]
                        ii = ie_loc[row0 + k, pl.ds(c * LANES, LANES)]
                        acc = acc + uu * ii
                    total = jnp.sum(acc)
                    out16 = jnp.where(lanes == k, total, out16)
                out_v[pl.ds(t0, LANES)] = out16

                tail = (uv >= TAILBASE) | (iv >= TAILBASE)

                @pl.when(jnp.any(tail))
                def _():
                    out16s = out_v[pl.ds(t0, LANES)]
                    for k in range(LANES):
                        ur = uv[k]
                        ir = iv[k]
                        cu = _splat(jnp.clip(ur - TAILBASE, 0, 63))
                        ci = _splat(jnp.clip(ir - TAILBASE, 0, 63))
                        acc = jnp.zeros((LANES,), jnp.float32)
                        for c in range(DIM // LANES):
                            cl = c * LANES + lanes
                            uu = ue_loc[row0 + k, pl.ds(c * LANES, LANES)]
                            ii = ie_loc[row0 + k, pl.ds(c * LANES, LANES)]
                            uu = jnp.where(
                                ur >= TAILBASE,
                                plsc.load_gather(tu_loc, [cl, cu]), uu)
                            ii = jnp.where(
                                ir >= TAILBASE,
                                plsc.load_gather(ti_loc, [cl, ci]), ii)
                            acc = acc + uu * ii
                        total = jnp.sum(acc)
                        out16s = jnp.where(lanes == k, total, out16s)
                    out_v[pl.ds(t0, LANES)] = out16s

                return carry

            lax.fori_loop(0, half // LANES, grp, 0)

        pltpu.sync_copy(out_v, out_hbm.at[pl.ds(base, b_per_w)])

    return dot_k(u, i, ue_buf, ie_buf, tail_u, tail_i)


def kernel(u, i, user_table, item_table):
    batch = u.shape[0]
    b_per_w = batch // NW
    u32 = u.astype(jnp.int32)
    i32 = i.astype(jnp.int32)
    iota = jnp.arange(batch, dtype=jnp.int32)
    ru, pu = lax.sort_key_val(u32, iota)
    ri, pi = lax.sort_key_val(i32, iota)

    def perm3(p):
        p4 = p.reshape(NW, b_per_w // 128, 128)
        return jnp.pad(p4, ((0, 0), (0, 8 - b_per_w // 128), (0, 0)))

    ut_t = user_table.T
    it_t = item_table.T
    tail_u = jnp.pad(user_table[TAILBASE:].T, ((0, 0), (0, 128 - (NROW - TAILBASE))))
    tail_i = jnp.pad(item_table[TAILBASE:].T, ((0, 0), (0, 128 - (NROW - TAILBASE))))

    ue_buf, ie_buf = _gather_rows_call(ru, perm3(pu), ut_t,
                                       ri, perm3(pi), it_t,
                                       b_per_w=b_per_w)
    return _dot_call(u32, i32, ue_buf, ie_buf, tail_u, tail_i,
                     b_per_w=b_per_w)
